# Initial kernel scaffold; baseline (speedup 1.0000x reference)
import jax
import jax.numpy as jnp
from jax.experimental import pallas as pl


def kernel(za, zb, pos, W1, b1, W2, b2):
    raise NotImplementedError("write your pallas kernel here")


# speedup vs baseline: 2.1081x; 2.1081x over previous
import functools

import jax
import jax.numpy as jnp
from jax.experimental import pallas as pl
from jax.experimental.pallas import tpu as pltpu

TAU = 0.5


def _mm_t(a, b):
    return jax.lax.dot_general(a, b, (((1,), (1,)), ((), ())),
                               preferred_element_type=jnp.float32)


def _proj_kernel(za_ref, zb_ref, w1_ref, b1_ref, w2_ref, b2_ref,
                 out_ref, an_ref, bn_ref, *, out_dim):
    w1 = w1_ref[...]
    b1 = b1_ref[...]
    w2 = w2_ref[...]
    b2 = b2_ref[...]

    def proj(x):
        h = _mm_t(x, w1) + b1
        h = jnp.where(h > 0, h, jnp.exp(h) - 1.0)
        return _mm_t(h, w2) + b2

    pa = proj(za_ref[...])
    pb = proj(zb_ref[...])
    out_ref[:, :out_dim] = pa
    out_ref[:, out_dim:] = pb
    na = jnp.sqrt(jnp.sum(pa * pa, axis=1, keepdims=True))
    nb = jnp.sqrt(jnp.sum(pb * pb, axis=1, keepdims=True))
    an_ref[...] = (pa / jnp.maximum(na, 1e-12)).astype(jnp.bfloat16)
    bn_ref[...] = (pb / jnp.maximum(nb, 1e-12)).astype(jnp.bfloat16)


def _main_kernel(a_ref, b_ref, pos_ref,
                 se_ref, sep_ref, ssp_ref, np_ref, *, tj):
    j = pl.program_id(1)

    @pl.when(j == 0)
    def _():
        se_ref[0, 0, 0] = 0.0
        sep_ref[0, 0, 0] = 0.0
        ssp_ref[0, 0, 0] = 0.0
        np_ref[0, 0, 0] = 0.0

    b = b_ref[pl.ds(j * tj, tj), :]
    s = _mm_t(a_ref[...], b) * (1.0 / TAU)
    e = jnp.exp(s)
    mf = pos_ref[...].astype(jnp.float32)
    se_ref[0, 0, 0] += jnp.sum(e)
    sep_ref[0, 0, 0] += jnp.sum(e * mf)
    ssp_ref[0, 0, 0] += jnp.sum(s * mf)
    np_ref[0, 0, 0] += jnp.sum(mf)


def _exact_kernel(neg_ref, a_ref, b_ref, pos_ref, vl_ref, *, tj):
    j = pl.program_id(1)

    @pl.when(j == 0)
    def _():
        vl_ref[0, 0, 0] = 0.0

    neg = neg_ref[0]
    b = b_ref[pl.ds(j * tj, tj), :]
    s = _mm_t(a_ref[...], b) * (1.0 / TAU)
    lv = jnp.log(jnp.exp(s) + neg)
    mf = pos_ref[...].astype(jnp.float32)
    vl_ref[0, 0, 0] += jnp.sum(lv * mf)


def kernel(za, zb, pos, W1, b1, W2, b2):
    n, hid = za.shape
    out_dim = W2.shape[0]
    tp = min(1024, n)
    ti = min(1024, n)
    tj = min(1024, n)
    gi, gj = n // ti, n // tj

    out, an, bn = pl.pallas_call(
        functools.partial(_proj_kernel, out_dim=out_dim),
        grid=(n // tp,),
        in_specs=[
            pl.BlockSpec((tp, hid), lambda t: (t, 0)),
            pl.BlockSpec((tp, hid), lambda t: (t, 0)),
            pl.BlockSpec((hid, hid), lambda t: (0, 0)),
            pl.BlockSpec((1, hid), lambda t: (0, 0)),
            pl.BlockSpec((out_dim, hid), lambda t: (0, 0)),
            pl.BlockSpec((1, out_dim), lambda t: (0, 0)),
        ],
        out_specs=[
            pl.BlockSpec((tp, 2 * out_dim), lambda t: (t, 0)),
            pl.BlockSpec((tp, out_dim), lambda t: (t, 0)),
            pl.BlockSpec((tp, out_dim), lambda t: (t, 0)),
        ],
        out_shape=[
            jax.ShapeDtypeStruct((n, 2 * out_dim), jnp.float32),
            jax.ShapeDtypeStruct((n, out_dim), jnp.bfloat16),
            jax.ShapeDtypeStruct((n, out_dim), jnp.bfloat16),
        ],
        compiler_params=pltpu.CompilerParams(
            dimension_semantics=("parallel",)),
    )(za, zb, W1, b1.reshape(1, hid), W2, b2.reshape(1, out_dim))

    se, sep, ssp, npn = pl.pallas_call(
        functools.partial(_main_kernel, tj=tj),
        grid=(gi, gj),
        in_specs=[
            pl.BlockSpec((ti, out_dim), lambda i, j: (i, 0)),
            pl.BlockSpec((n, out_dim), lambda i, j: (0, 0)),
            pl.BlockSpec((ti, tj), lambda i, j: (i, j)),
        ],
        out_specs=[
            pl.BlockSpec((1, 1, 1), lambda i, j: (i, 0, 0),
                         memory_space=pltpu.SMEM),
            pl.BlockSpec((1, 1, 1), lambda i, j: (i, 0, 0),
                         memory_space=pltpu.SMEM),
            pl.BlockSpec((1, 1, 1), lambda i, j: (i, 0, 0),
                         memory_space=pltpu.SMEM),
            pl.BlockSpec((1, 1, 1), lambda i, j: (i, 0, 0),
                         memory_space=pltpu.SMEM),
        ],
        out_shape=[
            jax.ShapeDtypeStruct((gi, 1, 1), jnp.float32),
            jax.ShapeDtypeStruct((gi, 1, 1), jnp.float32),
            jax.ShapeDtypeStruct((gi, 1, 1), jnp.float32),
            jax.ShapeDtypeStruct((gi, 1, 1), jnp.float32),
        ],
        compiler_params=pltpu.CompilerParams(
            dimension_semantics=("parallel", "arbitrary")),
    )(an, bn, pos)

    sum_e = jnp.sum(se)
    sum_e_pos = jnp.sum(sep)
    sum_s_pos = jnp.sum(ssp)
    n_pos_raw = jnp.sum(npn)
    n_pos = jnp.maximum(n_pos_raw, 1.0)
    neg_sum = sum_e - sum_e_pos

    def fast_loss(_):
        return (n_pos_raw * jnp.log(neg_sum)
                + sum_e_pos / neg_sum - sum_s_pos) / n_pos

    def exact_loss(_):
        vl = pl.pallas_call(
            functools.partial(_exact_kernel, tj=tj),
            grid=(gi, gj),
            in_specs=[
                pl.BlockSpec(memory_space=pltpu.SMEM),
                pl.BlockSpec((ti, out_dim), lambda i, j: (i, 0)),
                pl.BlockSpec((n, out_dim), lambda i, j: (0, 0)),
                pl.BlockSpec((ti, tj), lambda i, j: (i, j)),
            ],
            out_specs=pl.BlockSpec((1, 1, 1), lambda i, j: (i, 0, 0),
                                   memory_space=pltpu.SMEM),
            out_shape=jax.ShapeDtypeStruct((gi, 1, 1), jnp.float32),
            compiler_params=pltpu.CompilerParams(
                dimension_semantics=("parallel", "arbitrary")),
        )(jnp.maximum(neg_sum, 0.0).reshape(1), an, bn, pos)
        return (jnp.sum(vl) - sum_s_pos) / n_pos

    loss = jax.lax.cond(neg_sum >= 1e4, fast_loss, exact_loss, operand=None)
    return (loss, out)


# speedup vs baseline: 2.1777x; 1.0330x over previous
import functools

import jax
import jax.numpy as jnp
from jax.experimental import pallas as pl
from jax.experimental.pallas import tpu as pltpu

TAU = 0.5
LOG2E = 1.4426950408889634
LN2 = 0.6931471805599453


def _mm_t(a, b):
    return jax.lax.dot_general(a, b, (((1,), (1,)), ((), ())),
                               preferred_element_type=jnp.float32)


def _proj_kernel(za_ref, zb_ref, w1_ref, b1_ref, w2_ref, b2_ref,
                 out_ref, an_ref, bn_ref, *, out_dim):
    w1 = w1_ref[...]
    b1 = b1_ref[...]
    w2 = w2_ref[...]
    b2 = b2_ref[...]

    def proj(x):
        h = _mm_t(x, w1) + b1
        h = jnp.where(h > 0, h, jnp.exp(h) - 1.0)
        return _mm_t(h, w2) + b2

    pa = proj(za_ref[...])
    pb = proj(zb_ref[...])
    out_ref[:, :out_dim] = pa
    out_ref[:, out_dim:] = pb
    na = jnp.sqrt(jnp.sum(pa * pa, axis=1, keepdims=True))
    nb = jnp.sqrt(jnp.sum(pb * pb, axis=1, keepdims=True))
    an_ref[...] = (pa * (LOG2E / TAU) / jnp.maximum(na, 1e-12)
                   ).astype(jnp.bfloat16)
    bn_ref[...] = (pb / jnp.maximum(nb, 1e-12)).astype(jnp.bfloat16)


def _main_kernel(a_ref, b_ref, pos_ref,
                 neg_ref, ssp_ref, np_ref, *, tj):
    j = pl.program_id(1)

    @pl.when(j == 0)
    def _():
        neg_ref[0, 0, 0] = 0.0
        ssp_ref[0, 0, 0] = 0.0
        np_ref[0, 0, 0] = 0.0

    b = b_ref[pl.ds(j * tj, tj), :]
    sp = _mm_t(a_ref[...], b)
    e = jnp.exp2(sp)
    m = pos_ref[...]
    mf = m.astype(jnp.float32)
    neg_ref[0, 0, 0] += jnp.sum(jnp.where(m, 0.0, e))
    ssp_ref[0, 0, 0] += jnp.sum(sp * mf)
    np_ref[0, 0, 0] += jnp.sum(mf)


def _exact_kernel(neg_ref, a_ref, b_ref, pos_ref, vl_ref, *, tj):
    j = pl.program_id(1)

    @pl.when(j == 0)
    def _():
        vl_ref[0, 0, 0] = 0.0

    neg = neg_ref[0]
    b = b_ref[pl.ds(j * tj, tj), :]
    sp = _mm_t(a_ref[...], b)
    e = jnp.exp2(sp)
    lv = -jnp.log(e / (e + neg))
    mf = pos_ref[...].astype(jnp.float32)
    vl_ref[0, 0, 0] += jnp.sum(lv * mf)


def kernel(za, zb, pos, W1, b1, W2, b2):
    n, hid = za.shape
    out_dim = W2.shape[0]
    tp = min(1024, n)
    ti = min(1024, n)
    tj = min(1024, n)
    gi, gj = n // ti, n // tj

    out, an, bn = pl.pallas_call(
        functools.partial(_proj_kernel, out_dim=out_dim),
        grid=(n // tp,),
        in_specs=[
            pl.BlockSpec((tp, hid), lambda t: (t, 0)),
            pl.BlockSpec((tp, hid), lambda t: (t, 0)),
            pl.BlockSpec((hid, hid), lambda t: (0, 0)),
            pl.BlockSpec((1, hid), lambda t: (0, 0)),
            pl.BlockSpec((out_dim, hid), lambda t: (0, 0)),
            pl.BlockSpec((1, out_dim), lambda t: (0, 0)),
        ],
        out_specs=[
            pl.BlockSpec((tp, 2 * out_dim), lambda t: (t, 0)),
            pl.BlockSpec((tp, out_dim), lambda t: (t, 0)),
            pl.BlockSpec((tp, out_dim), lambda t: (t, 0)),
        ],
        out_shape=[
            jax.ShapeDtypeStruct((n, 2 * out_dim), jnp.float32),
            jax.ShapeDtypeStruct((n, out_dim), jnp.bfloat16),
            jax.ShapeDtypeStruct((n, out_dim), jnp.bfloat16),
        ],
        compiler_params=pltpu.CompilerParams(
            dimension_semantics=("parallel",)),
    )(za, zb, W1, b1.reshape(1, hid), W2, b2.reshape(1, out_dim))

    negp, ssp, npn = pl.pallas_call(
        functools.partial(_main_kernel, tj=tj),
        grid=(gi, gj),
        in_specs=[
            pl.BlockSpec((ti, out_dim), lambda i, j: (i, 0)),
            pl.BlockSpec((n, out_dim), lambda i, j: (0, 0)),
            pl.BlockSpec((ti, tj), lambda i, j: (i, j)),
        ],
        out_specs=[
            pl.BlockSpec((1, 1, 1), lambda i, j: (i, 0, 0),
                         memory_space=pltpu.SMEM),
            pl.BlockSpec((1, 1, 1), lambda i, j: (i, 0, 0),
                         memory_space=pltpu.SMEM),
            pl.BlockSpec((1, 1, 1), lambda i, j: (i, 0, 0),
                         memory_space=pltpu.SMEM),
        ],
        out_shape=[
            jax.ShapeDtypeStruct((gi, 1, 1), jnp.float32),
            jax.ShapeDtypeStruct((gi, 1, 1), jnp.float32),
            jax.ShapeDtypeStruct((gi, 1, 1), jnp.float32),
        ],
        compiler_params=pltpu.CompilerParams(
            dimension_semantics=("parallel", "arbitrary")),
    )(an, bn, pos)

    neg_sum = jnp.sum(negp)
    sum_s_pos = jnp.sum(ssp) * LN2
    n_pos_raw = jnp.sum(npn)
    n_pos = jnp.maximum(n_pos_raw, 1.0)

    def fast_loss(_):
        return (n_pos_raw * jnp.log(neg_sum) - sum_s_pos) / n_pos

    def exact_loss(_):
        vl = pl.pallas_call(
            functools.partial(_exact_kernel, tj=tj),
            grid=(gi, gj),
            in_specs=[
                pl.BlockSpec(memory_space=pltpu.SMEM),
                pl.BlockSpec((ti, out_dim), lambda i, j: (i, 0)),
                pl.BlockSpec((n, out_dim), lambda i, j: (0, 0)),
                pl.BlockSpec((ti, tj), lambda i, j: (i, j)),
            ],
            out_specs=pl.BlockSpec((1, 1, 1), lambda i, j: (i, 0, 0),
                                   memory_space=pltpu.SMEM),
            out_shape=jax.ShapeDtypeStruct((gi, 1, 1), jnp.float32),
            compiler_params=pltpu.CompilerParams(
                dimension_semantics=("parallel", "arbitrary")),
        )(jnp.maximum(neg_sum, 0.0).reshape(1), an, bn, pos)
        return jnp.sum(vl) / n_pos

    loss = jax.lax.cond(neg_sum >= 1e6, fast_loss, exact_loss, operand=None)
    return (loss, out)
